# baseline (speedup 1.0000x reference)
import functools

import numpy as np
import jax
import jax.numpy as jnp
from jax import lax
from jax.experimental import pallas as pl
from jax.experimental.pallas import tpu as pltpu
from jax.experimental.pallas import tpu_sc as plsc

L = 2048
F = 1025
FPAD = 1152
C = 128
B = 32
LEVELS_N = 3
K = 5
NSLOT = 16
BIG = FPAD

_HI = jax.lax.Precision.HIGHEST


def _build_bases():
    t = np.arange(L, dtype=np.int64)
    f = np.arange(FPAD, dtype=np.int64)
    ang = 2.0 * np.pi * ((f[:, None] * t[None, :]) % L).astype(np.float64) / L
    cosf = np.cos(ang)
    sinf = np.sin(ang)
    cosf[F:] = 0.0
    sinf[F:] = 0.0
    cos_tm = cosf.T.copy()
    sin_tm = sinf.T.copy()
    return (cosf.astype(np.float32), sinf.astype(np.float32),
            cos_tm.astype(np.float32), sin_tm.astype(np.float32))


_COSF, _SINF, _COS_TM, _SIN_TM = _build_bases()


def _topk_keep(mag):
    cnt = jnp.zeros((1, C), jnp.int32)
    keep = jnp.zeros((FPAD, C), jnp.bool_)
    cur = mag
    for _ in range(K):
        v = jnp.max(cur, axis=0, keepdims=True)
        eq = cur == v
        c = jnp.sum(eq.astype(jnp.int32), axis=0, keepdims=True)
        newcnt = cnt + c
        keep = jnp.logical_or(keep, jnp.logical_and(eq, newcnt < K))
        cnt = jnp.where(cnt < K, newcnt, cnt)
        cur = jnp.where(eq, -jnp.inf, cur)
    return keep


def _fwd_select_kernel(x_ref, cos_ref, sin_ref, idx_ref, w_ref, u_ref):
    x = x_ref[0]
    re = jnp.dot(cos_ref[...], x, precision=_HI,
                 preferred_element_type=jnp.float32)
    im = -jnp.dot(sin_ref[...], x, precision=_HI,
                  preferred_element_type=jnp.float32)
    mag = re * re + im * im
    lane = jax.lax.broadcasted_iota(jnp.int32, (FPAD, C), 1)
    mag = jnp.where(lane == 0, 0.0, mag)
    rowid = jax.lax.broadcasted_iota(jnp.int32, (FPAD, C), 0)
    zrow = jnp.zeros((1, C), jnp.float32)
    for lvl in range(LEVELS_N):
        keep = _topk_keep(mag)
        cand = jnp.where(keep, rowid, BIG)
        for j in range(4):
            m = jnp.min(cand, axis=0, keepdims=True)
            found = m < BIG
            sel = cand == m
            wj = jnp.sum(jnp.where(sel, re, 0.0), axis=0, keepdims=True)
            uj = jnp.sum(jnp.where(sel, im, 0.0), axis=0, keepdims=True)
            edge = jnp.logical_or(m == 0, m == L // 2)
            scl = jnp.where(edge, 1.0 / L, 2.0 / L)
            wj = jnp.where(found, wj * scl, 0.0)
            uj = jnp.where(jnp.logical_and(found, jnp.logical_not(edge)),
                           uj * (2.0 / L), 0.0)
            slot = lvl * 4 + j
            idx_ref[0, slot, :] = jnp.where(found, m, 0)[0]
            w_ref[0, slot, :] = wj[0]
            u_ref[0, slot, :] = uj[0]
            cand = jnp.where(sel, BIG, cand)
        mag = jnp.where(keep, 0.0, mag)
    for slot in range(LEVELS_N * 4, NSLOT):
        idx_ref[0, slot, :] = jnp.zeros((C,), jnp.int32)
        w_ref[0, slot, :] = zrow[0]
        u_ref[0, slot, :] = zrow[0]


TCH = 16
TSPAN = L // 32


def _sc_synth_body(cos_hbm, sin_hbm, idx_hbm, w_hbm, u_hbm, x_hbm,
                   s1, s2, s3, r1, r2, r3,
                   cosb, sinb, idxb, wb, ub, xb,
                   o1, o2, o3, o4, o5, o6):
    info = plsc.get_sparse_core_info()
    wid = lax.axis_index("s") * info.num_cores + lax.axis_index("c")
    t0 = wid * TSPAN
    souts = (o1, o2, o3)
    for tch in range(TSPAN // TCH):
        row0 = t0 + tch * TCH
        pltpu.sync_copy(cos_hbm.at[pl.ds(row0 * FPAD, TCH * FPAD)], cosb)
        pltpu.sync_copy(sin_hbm.at[pl.ds(row0 * FPAD, TCH * FPAD)], sinb)

        def b_body(b, _):
            pltpu.sync_copy(idx_hbm.at[b], idxb)
            pltpu.sync_copy(w_hbm.at[b], wb)
            pltpu.sync_copy(u_hbm.at[b], ub)
            pltpu.sync_copy(x_hbm.at[b, pl.ds(row0 * C, TCH * C)], xb)
            for lvl in range(LEVELS_N):
                ob = souts[lvl]

                def cc_body(cc, _, lvl=lvl, ob=ob):
                    col = cc * 16
                    acc = [jnp.zeros((16,), jnp.float32)
                           for _ in range(TCH)]
                    for j in range(4):
                        slot = lvl * 4 + j
                        iv = idxb[pl.ds(slot * C + col, 16)]
                        wv = wb[pl.ds(slot * C + col, 16)]
                        uv = ub[pl.ds(slot * C + col, 16)]
                        for trow in range(TCH):
                            fi = iv + trow * FPAD
                            cv = plsc.load_gather(cosb, [fi])
                            sv = plsc.load_gather(sinb, [fi])
                            acc[trow] = acc[trow] + wv * cv - uv * sv
                    for trow in range(TCH):
                        ob[pl.ds(trow * C + col, 16)] = acc[trow]
                    return 0

                lax.fori_loop(0, C // 16, cc_body, 0)

            def rc_body(cc, _):
                col = cc * 16
                for trow in range(TCH):
                    o = trow * C + col
                    xv = xb[pl.ds(o, 16)]
                    v1 = xv - o1[pl.ds(o, 16)]
                    o4[pl.ds(o, 16)] = v1
                    v2 = v1 - o2[pl.ds(o, 16)]
                    o5[pl.ds(o, 16)] = v2
                    o6[pl.ds(o, 16)] = v2 - o3[pl.ds(o, 16)]
                return 0

            lax.fori_loop(0, C // 16, rc_body, 0)
            for ob, oh in ((o1, s1), (o2, s2), (o3, s3),
                           (o4, r1), (o5, r2), (o6, r3)):
                pltpu.sync_copy(ob, oh.at[b, pl.ds(row0 * C, TCH * C)])
            return 0

        lax.fori_loop(0, B, b_body, 0)


def _make_sc_synth():
    mesh = plsc.VectorSubcoreMesh(core_axis_name="c", subcore_axis_name="s")
    out = jax.ShapeDtypeStruct((B, L * C), jnp.float32)
    return pl.kernel(
        _sc_synth_body,
        out_type=[out] * 6,
        mesh=mesh,
        compiler_params=pltpu.CompilerParams(use_tc_tiling_on_sc=False,
                                             needs_layout_passes=False),
        scratch_types=[
            pltpu.VMEM((TCH * FPAD,), jnp.float32),
            pltpu.VMEM((TCH * FPAD,), jnp.float32),
            pltpu.VMEM((NSLOT * C,), jnp.int32),
            pltpu.VMEM((NSLOT * C,), jnp.float32),
            pltpu.VMEM((NSLOT * C,), jnp.float32),
            pltpu.VMEM((TCH * C,), jnp.float32),
        ] + [pltpu.VMEM((TCH * C,), jnp.float32)] * 6,
    )


_SC_SYNTH = _make_sc_synth()


@jax.jit
def kernel(x):
    spec_x = pl.BlockSpec((1, L, C), lambda b: (b, 0, 0))
    spec_full_fl = pl.BlockSpec((FPAD, L), lambda b: (0, 0))
    spec_slot = pl.BlockSpec((1, NSLOT, C), lambda b: (b, 0, 0))

    idx, w, u = pl.pallas_call(
        _fwd_select_kernel,
        grid=(B,),
        in_specs=[spec_x, spec_full_fl, spec_full_fl],
        out_specs=[spec_slot] * 3,
        out_shape=[jax.ShapeDtypeStruct((B, NSLOT, C), jnp.int32),
                   jax.ShapeDtypeStruct((B, NSLOT, C), jnp.float32),
                   jax.ShapeDtypeStruct((B, NSLOT, C), jnp.float32)],
    )(x, _COSF, _SINF)

    outs = _SC_SYNTH(
        jnp.asarray(_COS_TM.reshape(-1)), jnp.asarray(_SIN_TM.reshape(-1)),
        idx.reshape(B, NSLOT * C), w.reshape(B, NSLOT * C),
        u.reshape(B, NSLOT * C), x.reshape(B, L * C))
    return tuple(o.reshape(B, L, C) for o in outs)


# speedup vs baseline: 1.2707x; 1.2707x over previous
import functools

import numpy as np
import jax
import jax.numpy as jnp
from jax import lax
from jax.experimental import pallas as pl
from jax.experimental.pallas import tpu as pltpu
from jax.experimental.pallas import tpu_sc as plsc

L = 2048
F = 1025
FPAD = 1152
C = 128
B = 32
LEVELS_N = 3
K = 5
NSLOT = 16
BIG = FPAD

_HI = jax.lax.Precision.HIGHEST


def _build_bases():
    t = np.arange(L, dtype=np.int64)
    f = np.arange(FPAD, dtype=np.int64)
    ang = 2.0 * np.pi * ((f[:, None] * t[None, :]) % L).astype(np.float64) / L
    cosf = np.cos(ang)
    sinf = np.sin(ang)
    cosf[F:] = 0.0
    sinf[F:] = 0.0
    cos_tm = cosf.T.copy()
    sin_tm = sinf.T.copy()
    return (cosf.astype(np.float32), sinf.astype(np.float32),
            cos_tm.astype(np.float32), sin_tm.astype(np.float32))


_COSF, _SINF, _COS_TM, _SIN_TM = _build_bases()


def _topk_keep(mag):
    cnt = jnp.zeros((1, C), jnp.int32)
    keep = jnp.zeros((FPAD, C), jnp.bool_)
    cur = mag
    for _ in range(K):
        v = jnp.max(cur, axis=0, keepdims=True)
        eq = cur == v
        c = jnp.sum(eq.astype(jnp.int32), axis=0, keepdims=True)
        newcnt = cnt + c
        keep = jnp.logical_or(keep, jnp.logical_and(eq, newcnt < K))
        cnt = jnp.where(cnt < K, newcnt, cnt)
        cur = jnp.where(eq, -jnp.inf, cur)
    return keep


def _fwd_select_kernel(x_ref, cos_ref, sin_ref, meta_ref):
    x = x_ref[0]
    re = jnp.dot(cos_ref[...], x, precision=_HI,
                 preferred_element_type=jnp.float32)
    im = -jnp.dot(sin_ref[...], x, precision=_HI,
                  preferred_element_type=jnp.float32)
    mag = re * re + im * im
    lane = jax.lax.broadcasted_iota(jnp.int32, (FPAD, C), 1)
    mag = jnp.where(lane == 0, 0.0, mag)
    rowid = jax.lax.broadcasted_iota(jnp.int32, (FPAD, C), 0)
    zrow = jnp.zeros((1, C), jnp.float32)
    for lvl in range(LEVELS_N):
        keep = _topk_keep(mag)
        cand = jnp.where(keep, rowid, BIG)
        for j in range(4):
            m = jnp.min(cand, axis=0, keepdims=True)
            found = m < BIG
            sel = cand == m
            wj = jnp.sum(jnp.where(sel, re, 0.0), axis=0, keepdims=True)
            uj = jnp.sum(jnp.where(sel, im, 0.0), axis=0, keepdims=True)
            edge = jnp.logical_or(m == 0, m == L // 2)
            scl = jnp.where(edge, 1.0 / L, 2.0 / L)
            wj = jnp.where(found, wj * scl, 0.0)
            uj = jnp.where(jnp.logical_and(found, jnp.logical_not(edge)),
                           uj * (2.0 / L), 0.0)
            slot = lvl * 4 + j
            meta_ref[0, slot, :] = jax.lax.bitcast_convert_type(
                jnp.where(found, m, 0), jnp.float32)[0]
            meta_ref[0, NSLOT + slot, :] = wj[0]
            meta_ref[0, 2 * NSLOT + slot, :] = uj[0]
            cand = jnp.where(sel, BIG, cand)
        mag = jnp.where(keep, 0.0, mag)
    zbits = jax.lax.bitcast_convert_type(jnp.zeros((C,), jnp.int32),
                                         jnp.float32)
    for slot in range(LEVELS_N * 4, NSLOT):
        meta_ref[0, slot, :] = zbits
        meta_ref[0, NSLOT + slot, :] = zrow[0]
        meta_ref[0, 2 * NSLOT + slot, :] = zrow[0]


TCH = 16
TSPAN = L // 32


def _sc_synth_body(cos_hbm, sin_hbm, meta_hbm, x_hbm,
                   s1, s2, s3, r1, r2, r3,
                   cosb, sinb, metab0, metab1, xb0, xb1,
                   ob0, ob1, isem0, isem1, osem0, osem1):
    info = plsc.get_sparse_core_info()
    wid = lax.axis_index("s") * info.num_cores + lax.axis_index("c")
    t0 = wid * TSPAN
    bufs = ((metab0, xb0, ob0, isem0, osem0),
            (metab1, xb1, ob1, isem1, osem1))
    outs_hbm = (s1, s2, s3, r1, r2, r3)

    for tch in range(TSPAN // TCH):
        row0 = t0 + tch * TCH
        pltpu.sync_copy(cos_hbm.at[pl.ds(row0 * FPAD, TCH * FPAD)], cosb)
        pltpu.sync_copy(sin_hbm.at[pl.ds(row0 * FPAD, TCH * FPAD)], sinb)

        def start_in(b, par):
            metab, xb, _, isem, _ = bufs[par]
            pltpu.make_async_copy(meta_hbm.at[b], metab, isem).start()
            pltpu.make_async_copy(
                x_hbm.at[b, pl.ds(row0 * C, TCH * C)], xb, isem).start()

        def wait_in(b, par):
            metab, xb, _, isem, _ = bufs[par]
            pltpu.make_async_copy(meta_hbm.at[b], metab, isem).wait()
            pltpu.make_async_copy(
                x_hbm.at[b, pl.ds(row0 * C, TCH * C)], xb, isem).wait()

        def start_out(b, par):
            _, _, ob, _, osem = bufs[par]
            for k in range(6):
                pltpu.make_async_copy(
                    ob.at[pl.ds(k * TCH * C, TCH * C)],
                    outs_hbm[k].at[b, pl.ds(row0 * C, TCH * C)],
                    osem).start()

        def wait_out(b, par):
            _, _, ob, _, osem = bufs[par]
            for k in range(6):
                pltpu.make_async_copy(
                    ob.at[pl.ds(k * TCH * C, TCH * C)],
                    outs_hbm[k].at[b, pl.ds(row0 * C, TCH * C)],
                    osem).wait()

        def compute(b, par):
            metab, xb, ob, _, _ = bufs[par]
            for lvl in range(LEVELS_N):

                def cc_body(cc, _, lvl=lvl):
                    col = cc * 16
                    acc = [jnp.zeros((16,), jnp.float32)
                           for _ in range(TCH)]
                    for j in range(4):
                        slot = lvl * 4 + j
                        iv = plsc.bitcast(
                            metab[pl.ds(slot * C + col, 16)], jnp.int32)
                        wv = metab[pl.ds((NSLOT + slot) * C + col, 16)]
                        uv = metab[pl.ds((2 * NSLOT + slot) * C + col, 16)]
                        for trow in range(TCH):
                            fi = iv + trow * FPAD
                            cv = plsc.load_gather(cosb, [fi])
                            sv = plsc.load_gather(sinb, [fi])
                            acc[trow] = acc[trow] + wv * cv - uv * sv
                    for trow in range(TCH):
                        ob[pl.ds(lvl * TCH * C + trow * C + col, 16)] = \
                            acc[trow]
                    return 0

                lax.fori_loop(0, C // 16, cc_body, 0)

            def rc_body(cc, _):
                col = cc * 16
                for trow in range(TCH):
                    o = trow * C + col
                    xv = xb[pl.ds(o, 16)]
                    v1 = xv - ob[pl.ds(o, 16)]
                    ob[pl.ds(3 * TCH * C + o, 16)] = v1
                    v2 = v1 - ob[pl.ds(TCH * C + o, 16)]
                    ob[pl.ds(4 * TCH * C + o, 16)] = v2
                    ob[pl.ds(5 * TCH * C + o, 16)] = \
                        v2 - ob[pl.ds(2 * TCH * C + o, 16)]
                return 0

            lax.fori_loop(0, C // 16, rc_body, 0)

        start_in(0, 0)

        def pair_body(i, _):
            b0 = 2 * i
            wait_in(b0, 0)
            start_in(b0 + 1, 1)

            @pl.when(i > 0)
            def _():
                wait_out(b0 - 2, 0)

            compute(b0, 0)
            start_out(b0, 0)

            wait_in(b0 + 1, 1)

            @pl.when(i < B // 2 - 1)
            def _():
                start_in(b0 + 2, 0)

            @pl.when(i > 0)
            def _():
                wait_out(b0 - 1, 1)

            compute(b0 + 1, 1)
            start_out(b0 + 1, 1)
            return 0

        lax.fori_loop(0, B // 2, pair_body, 0)
        wait_out(B - 2, 0)
        wait_out(B - 1, 1)


def _make_sc_synth():
    mesh = plsc.VectorSubcoreMesh(core_axis_name="c", subcore_axis_name="s")
    out = jax.ShapeDtypeStruct((B, L * C), jnp.float32)
    return pl.kernel(
        _sc_synth_body,
        out_type=[out] * 6,
        mesh=mesh,
        compiler_params=pltpu.CompilerParams(use_tc_tiling_on_sc=False,
                                             needs_layout_passes=False),
        scratch_types=[
            pltpu.VMEM((TCH * FPAD,), jnp.float32),
            pltpu.VMEM((TCH * FPAD,), jnp.float32),
            pltpu.VMEM((3 * NSLOT * C,), jnp.float32),
            pltpu.VMEM((3 * NSLOT * C,), jnp.float32),
            pltpu.VMEM((TCH * C,), jnp.float32),
            pltpu.VMEM((TCH * C,), jnp.float32),
            pltpu.VMEM((6 * TCH * C,), jnp.float32),
            pltpu.VMEM((6 * TCH * C,), jnp.float32),
            pltpu.SemaphoreType.DMA,
            pltpu.SemaphoreType.DMA,
            pltpu.SemaphoreType.DMA,
            pltpu.SemaphoreType.DMA,
        ],
    )


_SC_SYNTH = _make_sc_synth()


@jax.jit
def kernel(x):
    spec_x = pl.BlockSpec((1, L, C), lambda b: (b, 0, 0))
    spec_full_fl = pl.BlockSpec((FPAD, L), lambda b: (0, 0))
    spec_meta = pl.BlockSpec((1, 3 * NSLOT, C), lambda b: (b, 0, 0))

    meta = pl.pallas_call(
        _fwd_select_kernel,
        grid=(B,),
        in_specs=[spec_x, spec_full_fl, spec_full_fl],
        out_specs=spec_meta,
        out_shape=jax.ShapeDtypeStruct((B, 3 * NSLOT, C), jnp.float32),
    )(x, _COSF, _SINF)

    outs = _SC_SYNTH(
        jnp.asarray(_COS_TM.reshape(-1)), jnp.asarray(_SIN_TM.reshape(-1)),
        meta.reshape(B, 3 * NSLOT * C), x.reshape(B, L * C))
    return tuple(o.reshape(B, L, C) for o in outs)
